# baseline (speedup 1.0000x reference)
import functools

import jax
import jax.numpy as jnp
from jax import lax
from jax.experimental import pallas as pl
from jax.experimental.pallas import tpu as pltpu
from jax.experimental.pallas import tpu_sc as plsc

B = 4096
L = 200
DIM = 64
NW = 32
BPW = B // NW
NSLOT = 4
GATHER_SPLITS = ((0, 104), (104, 96))
LANES = 16

_mesh = plsc.VectorSubcoreMesh(core_axis_name="c", subcore_axis_name="s")


@functools.partial(
    pl.kernel,
    mesh=_mesh,
    compiler_params=pltpu.CompilerParams(use_tc_tiling_on_sc=False),
    out_type=jax.ShapeDtypeStruct((B, L, DIM), jnp.float32),
    scratch_types=[
        pltpu.VMEM((BPW, L), jnp.int32),
        pltpu.VMEM((NSLOT, L, DIM), jnp.float32),
        pltpu.VMEM((NSLOT, L, DIM), jnp.float32),
        pltpu.SemaphoreType.DMA,
        pltpu.SemaphoreType.DMA,
        pltpu.SemaphoreType.DMA,
        pltpu.SemaphoreType.DMA,
        pltpu.SemaphoreType.DMA,
        pltpu.SemaphoreType.DMA,
        pltpu.SemaphoreType.DMA,
        pltpu.SemaphoreType.DMA,
    ],
)
def _pe_kernel(x_hbm, idx_hbm, tab_hbm, out_hbm, idx_all, rows_v, xb_v,
               l0, l1, l2, l3, o0, o1, o2, o3):
    lsem = (l0, l1, l2, l3)
    osem = (o0, o1, o2, o3)
    wid = lax.axis_index("s") * 2 + lax.axis_index("c")
    b0 = wid * BPW

    pltpu.sync_copy(idx_hbm.at[pl.ds(b0, BPW)], idx_all)

    def load(c, s):
        for off, sz in GATHER_SPLITS:
            pltpu.async_copy(
                tab_hbm.at[idx_all.at[c, pl.ds(off, sz)]],
                rows_v.at[s, pl.ds(off, sz)],
                lsem[s],
            )
        pltpu.async_copy(x_hbm.at[b0 + c], xb_v.at[s], lsem[s])

    def wait_loads(s):
        pltpu.make_async_copy(x_hbm.at[b0], rows_v.at[s], lsem[s]).wait()
        pltpu.make_async_copy(x_hbm.at[b0], xb_v.at[s], lsem[s]).wait()

    def wait_out(s):
        pltpu.make_async_copy(xb_v.at[s], out_hbm.at[b0], osem[s]).wait()

    for s in range(NSLOT - 1):
        load(s, s)

    def chunk_group(p, carry):
        c0 = p * NSLOT
        for s in range(NSLOT):
            c = c0 + s
            wait_loads(s)

            def add_body(r, carry2):
                for k in range(DIM // LANES):
                    sl = pl.ds(k * LANES, LANES)
                    plsc.addupdate(xb_v.at[s, r, sl], rows_v[s, r, sl])
                return carry2

            lax.fori_loop(0, L, add_body, 0, unroll=4)
            pltpu.async_copy(xb_v.at[s], out_hbm.at[b0 + c], osem[s])

            cn = c + NSLOT - 1
            sn = (s + NSLOT - 1) % NSLOT

            @pl.when(cn < BPW)
            def _():
                @pl.when(cn >= NSLOT)
                def _():
                    wait_out(sn)

                load(cn, sn)

        return carry

    lax.fori_loop(0, BPW // NSLOT, chunk_group, 0)

    for s in range(NSLOT):
        wait_out(s)


def kernel(x, order, pos_enc):
    return _pe_kernel(x, order.astype(jnp.int32), pos_enc)
